# baseline (speedup 1.0000x reference)
import functools

import jax
import jax.numpy as jnp
from jax import lax
from jax.experimental import pallas as pl
from jax.experimental.pallas import tpu as pltpu
from jax.experimental.pallas import tpu_sc as plsc

VOCAB = 100000
BIGRAM_VOCAB = 1000000
BIGRAM_DIM = 32
MODEL_DIM = 768

NC = 2
NS = 16
NW = NC * NS
B_TOTAL = 4 * 4096
CHUNK = B_TOTAL // NW
HALF = CHUNK // 2
ROWS = BIGRAM_VOCAB // 4

_sc_mesh = plsc.VectorSubcoreMesh(
    core_axis_name="c", subcore_axis_name="s", num_cores=NC, num_subcores=NS
)


@functools.partial(
    pl.kernel,
    out_type=jax.ShapeDtypeStruct((2, BIGRAM_DIM, B_TOTAL), jnp.float32),
    mesh=_sc_mesh,
    compiler_params=pltpu.CompilerParams(
        use_tc_tiling_on_sc=True, needs_layout_passes=False),
    scratch_types=[
        pltpu.VMEM((CHUNK,), jnp.int32),
        pltpu.VMEM((CHUNK,), jnp.int32),
        pltpu.VMEM((CHUNK,), jnp.int32),
        pltpu.VMEM((CHUNK,), jnp.int32),
        pltpu.VMEM((CHUNK,), jnp.int32),
        pltpu.VMEM((CHUNK,), jnp.int32),
        pltpu.VMEM((HALF, 128), jnp.float32),
        pltpu.VMEM((HALF, 128), jnp.float32),
        pltpu.VMEM((BIGRAM_DIM, CHUNK), jnp.float32),
        pltpu.VMEM((BIGRAM_DIM, CHUNK), jnp.float32),
        pltpu.SemaphoreType.DMA,
        pltpu.SemaphoreType.DMA,
    ],
)
def _sc_gather(x_hbm, prev_hbm, t1_hbm, t2_hbm, out_hbm,
               x_v, prev_v, h1_v, h2_v, idx1_v, idx2_v,
               rawa_v, rawb_v, ex1_v, ex2_v, sema, semb):
    wid = lax.axis_index("s") * NC + lax.axis_index("c")
    base = wid * CHUNK
    pltpu.sync_copy(x_hbm.at[pl.ds(base, CHUNK)], x_v)
    pltpu.sync_copy(prev_hbm.at[pl.ds(base, CHUNK)], prev_v)

    def hashes(i, _):
        xa = x_v[pl.ds(i * 16, 16)]
        pa = prev_v[pl.ds(i * 16, 16)]
        h1 = (pa * 1024 + xa) % BIGRAM_VOCAB
        h2 = (pa + xa * 31) % BIGRAM_VOCAB
        h1_v[pl.ds(i * 16, 16)] = h1
        h2_v[pl.ds(i * 16, 16)] = h2
        idx1_v[pl.ds(i * 16, 16)] = h1 >> 2
        idx2_v[pl.ds(i * 16, 16)] = h2 >> 2
        return ()

    lax.fori_loop(0, CHUNK // 16, hashes, ())

    stages = [(idx1_v, h1_v, ex1_v, 0), (idx1_v, h1_v, ex1_v, 1),
              (idx2_v, h2_v, ex2_v, 0), (idx2_v, h2_v, ex2_v, 1)]
    tables = [t1_hbm, t1_hbm, t2_hbm, t2_hbm]
    bufs = [rawa_v, rawb_v]
    sems = [sema, semb]
    lane = lax.iota(jnp.int32, 16)

    def fire(s):
        idx_v, _, _, half = stages[s]
        buf, sem = bufs[s % 2], sems[s % 2]
        cps = []
        for c in range(2):
            cps.append(pltpu.async_copy(
                tables[s].at[idx_v.at[pl.ds(half * HALF + c * 128, 128)]],
                buf.at[pl.ds(c * 128, 128)], sem))
        return cps

    def extract(s):
        _, h_v, ex_v, half = stages[s]
        buf = bufs[s % 2]

        def body(g, _):
            hv = h_v[pl.ds(half * HALF + g * 16, 16)]
            sub = (hv & 3) * 32
            row = lane + g * 16
            for w in range(BIGRAM_DIM):
                vals = plsc.load_gather(buf, [row, sub + w])
                ex_v[w, pl.ds(half * HALF + g * 16, 16)] = vals
            return ()

        lax.fori_loop(0, HALF // 16, body, ())

    inflight = fire(0)
    for s in range(4):
        if s + 1 < 4:
            nxt = fire(s + 1)
        for cp in inflight:
            cp.wait()
        extract(s)
        if s + 1 < 4:
            inflight = nxt

    pltpu.sync_copy(ex1_v, out_hbm.at[0, :, pl.ds(base, CHUNK)])
    pltpu.sync_copy(ex2_v, out_hbm.at[1, :, pl.ds(base, CHUNK)])


M_BLK = 2048


def _tc_matmul_body(gate_ref, e_ref, w_ref, o_ref):
    g = jax.nn.sigmoid(gate_ref[0])
    w1 = w_ref[:, :BIGRAM_DIM] * g
    w2 = w_ref[:, BIGRAM_DIM:] * (1.0 - g)
    acc = lax.dot_general(e_ref[0], w1, (((0,), (1,)), ((), ())),
                          preferred_element_type=jnp.float32)
    acc += lax.dot_general(e_ref[1], w2, (((0,), (1,)), ((), ())),
                           preferred_element_type=jnp.float32)
    o_ref[...] = acc


def _tc_matmul(gate_flat, e_both, proj_w):
    return pl.pallas_call(
        _tc_matmul_body,
        grid=(B_TOTAL // M_BLK,),
        in_specs=[
            pl.BlockSpec(memory_space=pltpu.SMEM),
            pl.BlockSpec((2, BIGRAM_DIM, M_BLK), lambda i: (0, 0, i)),
            pl.BlockSpec((MODEL_DIM, 2 * BIGRAM_DIM), lambda i: (0, 0)),
        ],
        out_specs=pl.BlockSpec((M_BLK, MODEL_DIM), lambda i: (i, 0)),
        out_shape=jax.ShapeDtypeStruct((B_TOTAL, MODEL_DIM), jnp.float32),
    )(gate_flat, e_both, proj_w)


def kernel(x, embed1, embed2, proj_w, gate):
    batch, seqlen = x.shape
    x_flat = x.reshape(-1)
    prev_flat = jnp.pad(x[:, :-1], ((0, 0), (1, 0)), constant_values=0).reshape(-1)
    t1 = embed1.reshape(ROWS, 128)
    t2 = embed2.reshape(ROWS, 128)
    e_both = _sc_gather(x_flat, prev_flat, t1, t2)
    y = _tc_matmul(gate.reshape(-1), e_both, proj_w)
    return y.reshape(batch, seqlen, MODEL_DIM)


# speedup vs baseline: 2.7318x; 2.7318x over previous
import functools

import jax
import jax.numpy as jnp
from jax import lax
from jax.experimental import pallas as pl
from jax.experimental.pallas import tpu as pltpu
from jax.experimental.pallas import tpu_sc as plsc

VOCAB = 100000
BIGRAM_VOCAB = 1000000
BIGRAM_DIM = 32
MODEL_DIM = 768

NC = 2
NS = 16
NW = NC * NS
B_TOTAL = 4 * 4096
CHUNK = B_TOTAL // NW
HALF = CHUNK // 2
W_BLK = 8192
N_TBLK = -(-BIGRAM_VOCAB // W_BLK)
ROWS = N_TBLK * (W_BLK // 4)

_sc_mesh = plsc.VectorSubcoreMesh(
    core_axis_name="c", subcore_axis_name="s", num_cores=NC, num_subcores=NS
)


@functools.partial(
    pl.kernel,
    out_type=jax.ShapeDtypeStruct((2, BIGRAM_DIM, B_TOTAL), jnp.float32),
    mesh=_sc_mesh,
    compiler_params=pltpu.CompilerParams(
        use_tc_tiling_on_sc=True, needs_layout_passes=False),
    scratch_types=[
        pltpu.VMEM((CHUNK,), jnp.int32),
        pltpu.VMEM((CHUNK,), jnp.int32),
        pltpu.VMEM((CHUNK,), jnp.int32),
        pltpu.VMEM((CHUNK,), jnp.int32),
        pltpu.VMEM((CHUNK,), jnp.int32),
        pltpu.VMEM((CHUNK,), jnp.int32),
        pltpu.VMEM((HALF, 128), jnp.float32),
        pltpu.VMEM((HALF, 128), jnp.float32),
        pltpu.VMEM((BIGRAM_DIM, CHUNK), jnp.float32),
        pltpu.VMEM((BIGRAM_DIM, CHUNK), jnp.float32),
        pltpu.SemaphoreType.DMA,
        pltpu.SemaphoreType.DMA,
    ],
)
def _sc_gather(x_hbm, prev_hbm, t1_hbm, t2_hbm, out_hbm,
               x_v, prev_v, h1_v, h2_v, idx1_v, idx2_v,
               rawa_v, rawb_v, ex1_v, ex2_v, sema, semb):
    wid = lax.axis_index("s") * NC + lax.axis_index("c")
    base = wid * CHUNK
    pltpu.sync_copy(x_hbm.at[pl.ds(base, CHUNK)], x_v)
    pltpu.sync_copy(prev_hbm.at[pl.ds(base, CHUNK)], prev_v)

    def hashes(i, _):
        xa = x_v[pl.ds(i * 16, 16)]
        pa = prev_v[pl.ds(i * 16, 16)]
        h1 = (pa * 1024 + xa) % BIGRAM_VOCAB
        h2 = (pa + xa * 31) % BIGRAM_VOCAB
        h1_v[pl.ds(i * 16, 16)] = h1
        h2_v[pl.ds(i * 16, 16)] = h2
        idx1_v[pl.ds(i * 16, 16)] = ((h1 >> 13) << 11) | (h1 & 2047)
        idx2_v[pl.ds(i * 16, 16)] = ((h2 >> 13) << 11) | (h2 & 2047)
        return ()

    lax.fori_loop(0, CHUNK // 16, hashes, ())

    stages = [(idx1_v, h1_v, ex1_v, 0), (idx1_v, h1_v, ex1_v, 1),
              (idx2_v, h2_v, ex2_v, 0), (idx2_v, h2_v, ex2_v, 1)]
    tables = [t1_hbm, t1_hbm, t2_hbm, t2_hbm]
    bufs = [rawa_v, rawb_v]
    sems = [sema, semb]
    lane = lax.iota(jnp.int32, 16)

    def fire(s):
        idx_v, _, _, half = stages[s]
        buf, sem = bufs[s % 2], sems[s % 2]
        cps = []
        for c in range(2):
            cps.append(pltpu.async_copy(
                tables[s].at[idx_v.at[pl.ds(half * HALF + c * 128, 128)]],
                buf.at[pl.ds(c * 128, 128)], sem))
        return cps

    def extract(s):
        _, h_v, ex_v, half = stages[s]
        buf = bufs[s % 2]

        def body(g, _):
            hv = h_v[pl.ds(half * HALF + g * 16, 16)]
            sub = ((hv >> 11) & 3) * 32
            row = lane + g * 16
            for w in range(BIGRAM_DIM):
                vals = plsc.load_gather(buf, [row, sub + w])
                ex_v[w, pl.ds(half * HALF + g * 16, 16)] = vals
            return ()

        lax.fori_loop(0, HALF // 16, body, ())

    inflight = fire(0)
    for s in range(4):
        if s + 1 < 4:
            nxt = fire(s + 1)
        for cp in inflight:
            cp.wait()
        extract(s)
        if s + 1 < 4:
            inflight = nxt

    pltpu.sync_copy(ex1_v, out_hbm.at[0, :, pl.ds(base, CHUNK)])
    pltpu.sync_copy(ex2_v, out_hbm.at[1, :, pl.ds(base, CHUNK)])


def _tc_transpose_body(t_ref, o_ref):
    q = W_BLK // 4
    z = jnp.concatenate([t_ref[:, q * s:q * s + q] for s in range(4)], axis=0)
    o_ref[...] = z.T


def _tc_transpose(t_t):
    return pl.pallas_call(
        _tc_transpose_body,
        grid=(N_TBLK,),
        in_specs=[pl.BlockSpec((BIGRAM_DIM, W_BLK), lambda i: (0, i))],
        out_specs=pl.BlockSpec((W_BLK // 4, 128), lambda i: (i, 0)),
        out_shape=jax.ShapeDtypeStruct((ROWS, 128), jnp.float32),
    )(t_t)


M_BLK = 2048


def _tc_matmul_body(gate_ref, e_ref, w_ref, o_ref):
    g = jax.nn.sigmoid(gate_ref[0])
    w1 = w_ref[:, :BIGRAM_DIM] * g
    w2 = w_ref[:, BIGRAM_DIM:] * (1.0 - g)
    acc = lax.dot_general(e_ref[0], w1, (((0,), (1,)), ((), ())),
                          preferred_element_type=jnp.float32)
    acc += lax.dot_general(e_ref[1], w2, (((0,), (1,)), ((), ())),
                           preferred_element_type=jnp.float32)
    o_ref[...] = acc


def _tc_matmul(gate_flat, e_both, proj_w):
    return pl.pallas_call(
        _tc_matmul_body,
        grid=(B_TOTAL // M_BLK,),
        in_specs=[
            pl.BlockSpec(memory_space=pltpu.SMEM),
            pl.BlockSpec((2, BIGRAM_DIM, M_BLK), lambda i: (0, 0, i)),
            pl.BlockSpec((MODEL_DIM, 2 * BIGRAM_DIM), lambda i: (0, 0)),
        ],
        out_specs=pl.BlockSpec((M_BLK, MODEL_DIM), lambda i: (i, 0)),
        out_shape=jax.ShapeDtypeStruct((B_TOTAL, MODEL_DIM), jnp.float32),
    )(gate_flat, e_both, proj_w)


def kernel(x, embed1, embed2, proj_w, gate):
    batch, seqlen = x.shape
    x_flat = x.reshape(-1)
    prev_flat = jnp.pad(x[:, :-1], ((0, 0), (1, 0)), constant_values=0).reshape(-1)
    t1 = _tc_transpose(embed1.T)
    t2 = _tc_transpose(embed2.T)
    e_both = _sc_gather(x_flat, prev_flat, t1, t2)
    y = _tc_matmul(gate.reshape(-1), e_both, proj_w)
    return y.reshape(batch, seqlen, MODEL_DIM)


# speedup vs baseline: 2.8234x; 1.0335x over previous
import functools

import jax
import jax.numpy as jnp
from jax import lax
from jax.experimental import pallas as pl
from jax.experimental.pallas import tpu as pltpu
from jax.experimental.pallas import tpu_sc as plsc

VOCAB = 100000
BIGRAM_VOCAB = 1000000
BIGRAM_DIM = 32
MODEL_DIM = 768

NC = 2
NS = 16
NW = NC * NS
B_TOTAL = 4 * 4096
CHUNK = B_TOTAL // NW
HALF = CHUNK // 2
W_BLK = 8192
N_TBLK = -(-BIGRAM_VOCAB // W_BLK)
ROWS = N_TBLK * (W_BLK // 4)

_sc_mesh = plsc.VectorSubcoreMesh(
    core_axis_name="c", subcore_axis_name="s", num_cores=NC, num_subcores=NS
)


def _make_sc_gather(hash_mul, hash_mul_prev):
    @functools.partial(
        pl.kernel,
        out_type=jax.ShapeDtypeStruct((BIGRAM_DIM, B_TOTAL), jnp.float32),
        mesh=_sc_mesh,
        compiler_params=pltpu.CompilerParams(
            use_tc_tiling_on_sc=True, needs_layout_passes=False),
        scratch_types=[
            pltpu.VMEM((CHUNK,), jnp.int32),
            pltpu.VMEM((CHUNK,), jnp.int32),
            pltpu.VMEM((CHUNK,), jnp.int32),
            pltpu.VMEM((CHUNK,), jnp.int32),
            pltpu.VMEM((HALF, 128), jnp.float32),
            pltpu.VMEM((HALF, 128), jnp.float32),
            pltpu.VMEM((BIGRAM_DIM, CHUNK), jnp.float32),
            pltpu.SemaphoreType.DMA,
            pltpu.SemaphoreType.DMA,
        ],
    )
    def _sc_gather(x_hbm, prev_hbm, t_hbm, out_hbm,
                   x_v, prev_v, h_v, idx_v, rawa_v, rawb_v, ex_v, sema, semb):
        base = (lax.axis_index("s") * NC + lax.axis_index("c")) * CHUNK
        pltpu.sync_copy(x_hbm.at[pl.ds(base, CHUNK)], x_v)
        pltpu.sync_copy(prev_hbm.at[pl.ds(base, CHUNK)], prev_v)

        def hashes(i, _):
            xa = x_v[pl.ds(i * 16, 16)]
            pa = prev_v[pl.ds(i * 16, 16)]
            h = (pa * hash_mul_prev + xa * hash_mul) % BIGRAM_VOCAB
            h_v[pl.ds(i * 16, 16)] = h
            idx_v[pl.ds(i * 16, 16)] = ((h >> 13) << 11) | (h & 2047)
            return ()

        lax.fori_loop(0, CHUNK // 16, hashes, ())

        bufs = [rawa_v, rawb_v]
        sems = [sema, semb]
        lane = lax.iota(jnp.int32, 16)

        def fire(s):
            return [pltpu.async_copy(
                t_hbm.at[idx_v.at[pl.ds(s * HALF + c * 128, 128)]],
                bufs[s].at[pl.ds(c * 128, 128)], sems[s])
                for c in range(HALF // 128)]

        def extract(s):
            def body(g, _):
                hv = h_v[pl.ds(s * HALF + g * 16, 16)]
                sub = ((hv >> 11) & 3) * 32
                row = lane + g * 16
                for w in range(BIGRAM_DIM):
                    vals = plsc.load_gather(bufs[s], [row, sub + w])
                    ex_v[w, pl.ds(s * HALF + g * 16, 16)] = vals
                return ()

            lax.fori_loop(0, HALF // 16, body, ())

        inflight = fire(0)
        nxt = fire(1)
        for cp in inflight:
            cp.wait()
        extract(0)
        for cp in nxt:
            cp.wait()
        extract(1)
        pltpu.sync_copy(ex_v, out_hbm.at[:, pl.ds(base, CHUNK)])

    return _sc_gather


_sc_gather_1 = _make_sc_gather(hash_mul=1, hash_mul_prev=1024)
_sc_gather_2 = _make_sc_gather(hash_mul=31, hash_mul_prev=1)


def _tc_transpose_body(t_ref, o_ref):
    q = W_BLK // 4
    z = jnp.concatenate([t_ref[:, q * s:q * s + q] for s in range(4)], axis=0)
    o_ref[...] = z.T


def _tc_transpose(t_t):
    return pl.pallas_call(
        _tc_transpose_body,
        grid=(N_TBLK,),
        in_specs=[pl.BlockSpec((BIGRAM_DIM, W_BLK), lambda i: (0, i))],
        out_specs=pl.BlockSpec((W_BLK // 4, 128), lambda i: (i, 0)),
        out_shape=jax.ShapeDtypeStruct((ROWS, 128), jnp.float32),
    )(t_t)


M_BLK = 2048


def _tc_matmul_body(gate_ref, e1_ref, e2_ref, w_ref, o_ref):
    g = jax.nn.sigmoid(gate_ref[0])
    w1 = w_ref[:, :BIGRAM_DIM] * g
    w2 = w_ref[:, BIGRAM_DIM:] * (1.0 - g)
    acc = lax.dot_general(e1_ref[...], w1, (((0,), (1,)), ((), ())),
                          preferred_element_type=jnp.float32)
    acc += lax.dot_general(e2_ref[...], w2, (((0,), (1,)), ((), ())),
                           preferred_element_type=jnp.float32)
    o_ref[...] = acc


def _tc_matmul(gate_flat, e1, e2, proj_w):
    return pl.pallas_call(
        _tc_matmul_body,
        grid=(B_TOTAL // M_BLK,),
        in_specs=[
            pl.BlockSpec(memory_space=pltpu.SMEM),
            pl.BlockSpec((BIGRAM_DIM, M_BLK), lambda i: (0, i)),
            pl.BlockSpec((BIGRAM_DIM, M_BLK), lambda i: (0, i)),
            pl.BlockSpec((MODEL_DIM, 2 * BIGRAM_DIM), lambda i: (0, 0)),
        ],
        out_specs=pl.BlockSpec((M_BLK, MODEL_DIM), lambda i: (i, 0)),
        out_shape=jax.ShapeDtypeStruct((B_TOTAL, MODEL_DIM), jnp.float32),
    )(gate_flat, e1, e2, proj_w)


def kernel(x, embed1, embed2, proj_w, gate):
    batch, seqlen = x.shape
    x_flat = x.reshape(-1)
    prev_flat = jnp.pad(x[:, :-1], ((0, 0), (1, 0)), constant_values=0).reshape(-1)
    t1 = _tc_transpose(embed1.T)
    e1 = _sc_gather_1(x_flat, prev_flat, t1)
    t2 = _tc_transpose(embed2.T)
    e2 = _sc_gather_2(x_flat, prev_flat, t2)
    y = _tc_matmul(gate.reshape(-1), e1, e2, proj_w)
    return y.reshape(batch, seqlen, MODEL_DIM)


# speedup vs baseline: 3.2835x; 1.1630x over previous
import functools

import jax
import jax.numpy as jnp
from jax import lax
from jax.experimental import pallas as pl
from jax.experimental.pallas import tpu as pltpu
from jax.experimental.pallas import tpu_sc as plsc

VOCAB = 100000
BIGRAM_VOCAB = 1000000
BIGRAM_DIM = 32
MODEL_DIM = 768

NC = 2
NS = 16
NW = NC * NS
B_TOTAL = 4 * 4096
CHUNK = B_TOTAL // NW
HALF = CHUNK // 2
W_BLK = 8192
C_BLK = W_BLK // 8
N_TBLK = -(-BIGRAM_VOCAB // W_BLK)
ROWS = N_TBLK * C_BLK

_sc_mesh = plsc.VectorSubcoreMesh(
    core_axis_name="c", subcore_axis_name="s", num_cores=NC, num_subcores=NS
)


def _make_sc_gather(hash_mul, hash_mul_prev):
    @functools.partial(
        pl.kernel,
        out_type=jax.ShapeDtypeStruct((BIGRAM_DIM, B_TOTAL), jnp.float32),
        mesh=_sc_mesh,
        compiler_params=pltpu.CompilerParams(
            use_tc_tiling_on_sc=True, needs_layout_passes=False),
        scratch_types=[
            pltpu.VMEM((CHUNK,), jnp.int32),
            pltpu.VMEM((CHUNK,), jnp.int32),
            pltpu.VMEM((CHUNK,), jnp.int32),
            pltpu.VMEM((CHUNK,), jnp.int32),
            pltpu.VMEM((HALF, 128), jnp.int32),
            pltpu.VMEM((HALF, 128), jnp.int32),
            pltpu.VMEM((BIGRAM_DIM, CHUNK), jnp.float32),
            pltpu.SemaphoreType.DMA,
            pltpu.SemaphoreType.DMA,
        ],
    )
    def _sc_gather(x_hbm, prev_hbm, t_hbm, out_hbm,
                   x_v, prev_v, h_v, idx_v, rawa_v, rawb_v, ex_v, sema, semb):
        base = (lax.axis_index("s") * NC + lax.axis_index("c")) * CHUNK
        pltpu.sync_copy(x_hbm.at[pl.ds(base, CHUNK)], x_v)
        pltpu.sync_copy(prev_hbm.at[pl.ds(base, CHUNK)], prev_v)

        def hashes(i, _):
            xa = x_v[pl.ds(i * 16, 16)]
            pa = prev_v[pl.ds(i * 16, 16)]
            h = (pa * hash_mul_prev + xa * hash_mul) % BIGRAM_VOCAB
            h_v[pl.ds(i * 16, 16)] = h
            idx_v[pl.ds(i * 16, 16)] = ((h >> 13) << 10) | (h & 1023)
            return ()

        lax.fori_loop(0, CHUNK // 16, hashes, ())

        bufs = [rawa_v, rawb_v]
        sems = [sema, semb]
        lane = lax.iota(jnp.int32, 16)

        def fire(s):
            return [pltpu.async_copy(
                t_hbm.at[idx_v.at[pl.ds(s * HALF + c * 128, 128)]],
                bufs[s].at[pl.ds(c * 128, 128)], sems[s])
                for c in range(HALF // 128)]

        def extract(s):
            def body(g, _):
                hv = h_v[pl.ds(s * HALF + g * 16, 16)]
                sub = ((hv >> 10) & 3) * 32
                is_hi = ((hv >> 10) & 7) < 4
                row = lane + g * 16
                for w in range(BIGRAM_DIM):
                    w32 = plsc.load_gather(bufs[s], [row, sub + w])
                    bits = jnp.where(is_hi, w32 & jnp.int32(-65536), w32 << 16)
                    ex_v[w, pl.ds(s * HALF + g * 16, 16)] = plsc.bitcast(
                        bits, jnp.float32)
                return ()

            lax.fori_loop(0, HALF // 16, body, ())

        inflight = fire(0)
        nxt = fire(1)
        for cp in inflight:
            cp.wait()
        extract(0)
        for cp in nxt:
            cp.wait()
        extract(1)
        pltpu.sync_copy(ex_v, out_hbm.at[:, pl.ds(base, CHUNK)])

    return _sc_gather


_sc_gather_1 = _make_sc_gather(hash_mul=1, hash_mul_prev=1024)
_sc_gather_2 = _make_sc_gather(hash_mul=31, hash_mul_prev=1)


def _tc_transpose_body(t_ref, o_ref):
    z = jnp.concatenate(
        [t_ref[:, C_BLK * s:C_BLK * s + C_BLK] for s in range(8)], axis=0)
    bits = lax.bitcast_convert_type(
        z.astype(jnp.bfloat16), jnp.uint16).astype(jnp.uint32)
    zi = (bits[:128] << 16) | bits[128:]
    o_ref[...] = lax.bitcast_convert_type(zi, jnp.int32).T


def _tc_transpose(t_t):
    return pl.pallas_call(
        _tc_transpose_body,
        grid=(N_TBLK,),
        in_specs=[pl.BlockSpec((BIGRAM_DIM, W_BLK), lambda i: (0, i))],
        out_specs=pl.BlockSpec((C_BLK, 128), lambda i: (i, 0)),
        out_shape=jax.ShapeDtypeStruct((ROWS, 128), jnp.int32),
    )(t_t)


M_BLK = 2048


def _tc_matmul_body(gate_ref, e1_ref, e2_ref, w_ref, o_ref):
    g = jax.nn.sigmoid(gate_ref[0])
    w1 = w_ref[:, :BIGRAM_DIM] * g
    w2 = w_ref[:, BIGRAM_DIM:] * (1.0 - g)
    acc = lax.dot_general(e1_ref[...], w1, (((0,), (1,)), ((), ())),
                          preferred_element_type=jnp.float32)
    acc += lax.dot_general(e2_ref[...], w2, (((0,), (1,)), ((), ())),
                           preferred_element_type=jnp.float32)
    o_ref[...] = acc


def _tc_matmul(gate_flat, e1, e2, proj_w):
    return pl.pallas_call(
        _tc_matmul_body,
        grid=(B_TOTAL // M_BLK,),
        in_specs=[
            pl.BlockSpec(memory_space=pltpu.SMEM),
            pl.BlockSpec((BIGRAM_DIM, M_BLK), lambda i: (0, i)),
            pl.BlockSpec((BIGRAM_DIM, M_BLK), lambda i: (0, i)),
            pl.BlockSpec((MODEL_DIM, 2 * BIGRAM_DIM), lambda i: (0, 0)),
        ],
        out_specs=pl.BlockSpec((M_BLK, MODEL_DIM), lambda i: (i, 0)),
        out_shape=jax.ShapeDtypeStruct((B_TOTAL, MODEL_DIM), jnp.float32),
    )(gate_flat, e1, e2, proj_w)


def kernel(x, embed1, embed2, proj_w, gate):
    batch, seqlen = x.shape
    x_flat = x.reshape(-1)
    prev_flat = jnp.pad(x[:, :-1], ((0, 0), (1, 0)), constant_values=0).reshape(-1)
    t1 = _tc_transpose(embed1.T)
    e1 = _sc_gather_1(x_flat, prev_flat, t1)
    t2 = _tc_transpose(embed2.T)
    e2 = _sc_gather_2(x_flat, prev_flat, t2)
    y = _tc_matmul(gate.reshape(-1), e1, e2, proj_w)
    return y.reshape(batch, seqlen, MODEL_DIM)


# speedup vs baseline: 4.1963x; 1.2780x over previous
import functools

import jax
import jax.numpy as jnp
from jax import lax
from jax.experimental import pallas as pl
from jax.experimental.pallas import tpu as pltpu
from jax.experimental.pallas import tpu_sc as plsc

VOCAB = 100000
BIGRAM_VOCAB = 1000000
BIGRAM_DIM = 32
MODEL_DIM = 768

NC = 2
NS = 16
NW = NC * NS
B_TOTAL = 4 * 4096
CHUNK = B_TOTAL // NW
HALF = CHUNK // 2
W_BLK = 16384
C_BLK = W_BLK // 8
N_TBLK = -(-BIGRAM_VOCAB // W_BLK)
ROWS = N_TBLK * C_BLK

_sc_mesh = plsc.VectorSubcoreMesh(
    core_axis_name="c", subcore_axis_name="s", num_cores=NC, num_subcores=NS
)


def _make_sc_gather(hash_mul, hash_mul_prev):
    @functools.partial(
        pl.kernel,
        out_type=jax.ShapeDtypeStruct((BIGRAM_DIM, B_TOTAL), jnp.float32),
        mesh=_sc_mesh,
        compiler_params=pltpu.CompilerParams(
            use_tc_tiling_on_sc=True, needs_layout_passes=False),
        scratch_types=[
            pltpu.VMEM((CHUNK,), jnp.int32),
            pltpu.VMEM((CHUNK,), jnp.int32),
            pltpu.VMEM((CHUNK,), jnp.int32),
            pltpu.VMEM((CHUNK,), jnp.int32),
            pltpu.VMEM((HALF, 128), jnp.int32),
            pltpu.VMEM((HALF, 128), jnp.int32),
            pltpu.VMEM((BIGRAM_DIM, CHUNK), jnp.float32),
            pltpu.SemaphoreType.DMA,
            pltpu.SemaphoreType.DMA,
        ],
    )
    def _sc_gather(x_hbm, prev_hbm, t_hbm, out_hbm,
                   x_v, prev_v, h_v, idx_v, rawa_v, rawb_v, ex_v, sema, semb):
        base = (lax.axis_index("s") * NC + lax.axis_index("c")) * CHUNK
        pltpu.sync_copy(x_hbm.at[pl.ds(base, CHUNK)], x_v)
        pltpu.sync_copy(prev_hbm.at[pl.ds(base, CHUNK)], prev_v)

        def hashes(i, _):
            xa = x_v[pl.ds(i * 16, 16)]
            pa = prev_v[pl.ds(i * 16, 16)]
            h = (pa * hash_mul_prev + xa * hash_mul) % BIGRAM_VOCAB
            h_v[pl.ds(i * 16, 16)] = h
            idx_v[pl.ds(i * 16, 16)] = ((h >> 14) << 11) | (h & 2047)
            return ()

        lax.fori_loop(0, CHUNK // 16, hashes, ())

        bufs = [rawa_v, rawb_v]
        sems = [sema, semb]
        lane = lax.iota(jnp.int32, 16)

        def fire(s):
            return [pltpu.async_copy(
                t_hbm.at[idx_v.at[pl.ds(s * HALF + c * 128, 128)]],
                bufs[s].at[pl.ds(c * 128, 128)], sems[s])
                for c in range(HALF // 128)]

        def extract(s):
            def body(g, _):
                hv = h_v[pl.ds(s * HALF + g * 16, 16)]
                sub = ((hv >> 11) & 3) * 32
                is_hi = ((hv >> 11) & 7) < 4
                row = lane + g * 16
                for w in range(BIGRAM_DIM):
                    w32 = plsc.load_gather(bufs[s], [row, sub + w])
                    bits = jnp.where(is_hi, w32 & jnp.int32(-65536), w32 << 16)
                    ex_v[w, pl.ds(s * HALF + g * 16, 16)] = plsc.bitcast(
                        bits, jnp.float32)
                return ()

            lax.fori_loop(0, HALF // 16, body, ())

        inflight = fire(0)
        nxt = fire(1)
        for cp in inflight:
            cp.wait()
        extract(0)
        for cp in nxt:
            cp.wait()
        extract(1)
        pltpu.sync_copy(ex_v, out_hbm.at[:, pl.ds(base, CHUNK)])

    return _sc_gather


_sc_gather_1 = _make_sc_gather(hash_mul=1, hash_mul_prev=1024)
_sc_gather_2 = _make_sc_gather(hash_mul=31, hash_mul_prev=1)


def _tc_transpose_body(t_ref, o_ref):
    z = jnp.concatenate(
        [t_ref[:, C_BLK * s:C_BLK * s + C_BLK] for s in range(8)], axis=0)
    bits = lax.bitcast_convert_type(
        z.astype(jnp.bfloat16), jnp.uint16).astype(jnp.uint32)
    zi = (bits[:128] << 16) | bits[128:]
    o_ref[...] = lax.bitcast_convert_type(zi, jnp.int32).T


def _tc_transpose(t_t):
    return pl.pallas_call(
        _tc_transpose_body,
        grid=(N_TBLK,),
        in_specs=[pl.BlockSpec((BIGRAM_DIM, W_BLK), lambda i: (0, i))],
        out_specs=pl.BlockSpec((C_BLK, 128), lambda i: (i, 0)),
        out_shape=jax.ShapeDtypeStruct((ROWS, 128), jnp.int32),
    )(t_t)


M_BLK = 2048


def _tc_matmul_body(gate_ref, e1_ref, e2_ref, w_ref, o_ref):
    g = jax.nn.sigmoid(gate_ref[0])
    w1 = w_ref[:, :BIGRAM_DIM] * g
    w2 = w_ref[:, BIGRAM_DIM:] * (1.0 - g)
    acc = lax.dot_general(e1_ref[...], w1, (((0,), (1,)), ((), ())),
                          preferred_element_type=jnp.float32)
    acc += lax.dot_general(e2_ref[...], w2, (((0,), (1,)), ((), ())),
                           preferred_element_type=jnp.float32)
    o_ref[...] = acc


def _tc_matmul(gate_flat, e1, e2, proj_w):
    return pl.pallas_call(
        _tc_matmul_body,
        grid=(B_TOTAL // M_BLK,),
        in_specs=[
            pl.BlockSpec(memory_space=pltpu.SMEM),
            pl.BlockSpec((BIGRAM_DIM, M_BLK), lambda i: (0, i)),
            pl.BlockSpec((BIGRAM_DIM, M_BLK), lambda i: (0, i)),
            pl.BlockSpec((MODEL_DIM, 2 * BIGRAM_DIM), lambda i: (0, 0)),
        ],
        out_specs=pl.BlockSpec((M_BLK, MODEL_DIM), lambda i: (i, 0)),
        out_shape=jax.ShapeDtypeStruct((B_TOTAL, MODEL_DIM), jnp.float32),
    )(gate_flat, e1, e2, proj_w)


def kernel(x, embed1, embed2, proj_w, gate):
    batch, seqlen = x.shape
    x_flat = x.reshape(-1)
    prev_flat = jnp.pad(x[:, :-1], ((0, 0), (1, 0)), constant_values=0).reshape(-1)
    t1 = _tc_transpose(embed1.T)
    e1 = _sc_gather_1(x_flat, prev_flat, t1)
    t2 = _tc_transpose(embed2.T)
    e2 = _sc_gather_2(x_flat, prev_flat, t2)
    y = _tc_matmul(gate.reshape(-1), e1, e2, proj_w)
    return y.reshape(batch, seqlen, MODEL_DIM)


# speedup vs baseline: 4.8813x; 1.1632x over previous
import functools

import jax
import jax.numpy as jnp
from jax import lax
from jax.experimental import pallas as pl
from jax.experimental.pallas import tpu as pltpu
from jax.experimental.pallas import tpu_sc as plsc

VOCAB = 100000
BIGRAM_VOCAB = 1000000
BIGRAM_DIM = 32
MODEL_DIM = 768

NC = 2
NS = 16
NW = NC * NS
B_TOTAL = 4 * 4096
CHUNK = B_TOTAL // NW
HALF = CHUNK // 2
W_BLK = 32768
C_BLK = W_BLK // 8
N_TBLK = -(-BIGRAM_VOCAB // W_BLK)
ROWS = N_TBLK * C_BLK

_sc_mesh = plsc.VectorSubcoreMesh(
    core_axis_name="c", subcore_axis_name="s", num_cores=NC, num_subcores=NS
)


def _make_sc_gather(hash_mul, hash_mul_prev):
    @functools.partial(
        pl.kernel,
        out_type=jax.ShapeDtypeStruct((BIGRAM_DIM, B_TOTAL), jnp.float32),
        mesh=_sc_mesh,
        compiler_params=pltpu.CompilerParams(
            use_tc_tiling_on_sc=True, needs_layout_passes=False),
        scratch_types=[
            pltpu.VMEM((CHUNK,), jnp.int32),
            pltpu.VMEM((CHUNK,), jnp.int32),
            pltpu.VMEM((CHUNK,), jnp.int32),
            pltpu.VMEM((CHUNK,), jnp.int32),
            pltpu.VMEM((HALF, 128), jnp.int32),
            pltpu.VMEM((HALF, 128), jnp.int32),
            pltpu.VMEM((BIGRAM_DIM, CHUNK), jnp.float32),
            pltpu.SemaphoreType.DMA,
            pltpu.SemaphoreType.DMA,
        ],
    )
    def _sc_gather(x_hbm, prev_hbm, t_hbm, out_hbm,
                   x_v, prev_v, h_v, idx_v, rawa_v, rawb_v, ex_v, sema, semb):
        base = (lax.axis_index("s") * NC + lax.axis_index("c")) * CHUNK
        pltpu.sync_copy(x_hbm.at[pl.ds(base, CHUNK)], x_v)
        pltpu.sync_copy(prev_hbm.at[pl.ds(base, CHUNK)], prev_v)

        def hashes(i, _):
            xa = x_v[pl.ds(i * 16, 16)]
            pa = prev_v[pl.ds(i * 16, 16)]
            h = (pa * hash_mul_prev + xa * hash_mul) % BIGRAM_VOCAB
            h_v[pl.ds(i * 16, 16)] = h
            idx_v[pl.ds(i * 16, 16)] = ((h >> 15) << 12) | (h & 4095)
            return ()

        lax.fori_loop(0, CHUNK // 16, hashes, ())

        bufs = [rawa_v, rawb_v]
        sems = [sema, semb]
        lane = lax.iota(jnp.int32, 16)

        def fire(s):
            return [pltpu.async_copy(
                t_hbm.at[idx_v.at[pl.ds(s * HALF + c * 128, 128)]],
                bufs[s].at[pl.ds(c * 128, 128)], sems[s])
                for c in range(HALF // 128)]

        def extract(s):
            def body(g, _):
                hv = h_v[pl.ds(s * HALF + g * 16, 16)]
                sub = ((hv >> 12) & 3) * 32
                is_hi = ((hv >> 12) & 7) < 4
                row = lane + g * 16
                for w in range(BIGRAM_DIM):
                    w32 = plsc.load_gather(bufs[s], [row, sub + w])
                    bits = jnp.where(is_hi, w32 & jnp.int32(-65536), w32 << 16)
                    ex_v[w, pl.ds(s * HALF + g * 16, 16)] = plsc.bitcast(
                        bits, jnp.float32)
                return ()

            lax.fori_loop(0, HALF // 16, body, ())

        inflight = fire(0)
        nxt = fire(1)
        for cp in inflight:
            cp.wait()
        extract(0)
        for cp in nxt:
            cp.wait()
        extract(1)
        pltpu.sync_copy(ex_v, out_hbm.at[:, pl.ds(base, CHUNK)])

    return _sc_gather


_sc_gather_1 = _make_sc_gather(hash_mul=1, hash_mul_prev=1024)
_sc_gather_2 = _make_sc_gather(hash_mul=31, hash_mul_prev=1)


def _tc_transpose_body(t_ref, o_ref):
    z = jnp.concatenate(
        [t_ref[:, C_BLK * s:C_BLK * s + C_BLK] for s in range(8)], axis=0)
    bits = lax.bitcast_convert_type(
        z.astype(jnp.bfloat16), jnp.uint16).astype(jnp.uint32)
    zi = (bits[:128] << 16) | bits[128:]
    o_ref[...] = lax.bitcast_convert_type(zi, jnp.int32).T


def _tc_transpose(t_t):
    return pl.pallas_call(
        _tc_transpose_body,
        grid=(N_TBLK,),
        in_specs=[pl.BlockSpec((BIGRAM_DIM, W_BLK), lambda i: (0, i))],
        out_specs=pl.BlockSpec((C_BLK, 128), lambda i: (i, 0)),
        out_shape=jax.ShapeDtypeStruct((ROWS, 128), jnp.int32),
    )(t_t)


M_BLK = 2048


def _tc_matmul_body(gate_ref, e1_ref, e2_ref, w_ref, o_ref):
    g = jax.nn.sigmoid(gate_ref[0])
    w1 = w_ref[:, :BIGRAM_DIM] * g
    w2 = w_ref[:, BIGRAM_DIM:] * (1.0 - g)
    acc = lax.dot_general(e1_ref[...], w1, (((0,), (1,)), ((), ())),
                          preferred_element_type=jnp.float32)
    acc += lax.dot_general(e2_ref[...], w2, (((0,), (1,)), ((), ())),
                           preferred_element_type=jnp.float32)
    o_ref[...] = acc


def _tc_matmul(gate_flat, e1, e2, proj_w):
    return pl.pallas_call(
        _tc_matmul_body,
        grid=(B_TOTAL // M_BLK,),
        in_specs=[
            pl.BlockSpec(memory_space=pltpu.SMEM),
            pl.BlockSpec((BIGRAM_DIM, M_BLK), lambda i: (0, i)),
            pl.BlockSpec((BIGRAM_DIM, M_BLK), lambda i: (0, i)),
            pl.BlockSpec((MODEL_DIM, 2 * BIGRAM_DIM), lambda i: (0, 0)),
        ],
        out_specs=pl.BlockSpec((M_BLK, MODEL_DIM), lambda i: (i, 0)),
        out_shape=jax.ShapeDtypeStruct((B_TOTAL, MODEL_DIM), jnp.float32),
    )(gate_flat, e1, e2, proj_w)


def kernel(x, embed1, embed2, proj_w, gate):
    batch, seqlen = x.shape
    x_flat = x.reshape(-1)
    prev_flat = jnp.pad(x[:, :-1], ((0, 0), (1, 0)), constant_values=0).reshape(-1)
    t1 = _tc_transpose(embed1.T)
    e1 = _sc_gather_1(x_flat, prev_flat, t1)
    t2 = _tc_transpose(embed2.T)
    e2 = _sc_gather_2(x_flat, prev_flat, t2)
    y = _tc_matmul(gate.reshape(-1), e1, e2, proj_w)
    return y.reshape(batch, seqlen, MODEL_DIM)


# speedup vs baseline: 4.9997x; 1.0243x over previous
import functools

import jax
import jax.numpy as jnp
from jax import lax
from jax.experimental import pallas as pl
from jax.experimental.pallas import tpu as pltpu
from jax.experimental.pallas import tpu_sc as plsc

VOCAB = 100000
BIGRAM_VOCAB = 1000000
BIGRAM_DIM = 32
MODEL_DIM = 768

NC = 2
NS = 16
NW = NC * NS
B_TOTAL = 4 * 4096
CHUNK = B_TOTAL // NW
HALF = CHUNK // 2
W_BLK = 65536
C_BLK = W_BLK // 8
N_TBLK = -(-BIGRAM_VOCAB // W_BLK)
ROWS = N_TBLK * C_BLK

_sc_mesh = plsc.VectorSubcoreMesh(
    core_axis_name="c", subcore_axis_name="s", num_cores=NC, num_subcores=NS
)


def _make_sc_gather(hash_mul, hash_mul_prev):
    @functools.partial(
        pl.kernel,
        out_type=jax.ShapeDtypeStruct((BIGRAM_DIM, B_TOTAL), jnp.float32),
        mesh=_sc_mesh,
        compiler_params=pltpu.CompilerParams(
            use_tc_tiling_on_sc=True, needs_layout_passes=False),
        scratch_types=[
            pltpu.VMEM((CHUNK,), jnp.int32),
            pltpu.VMEM((CHUNK,), jnp.int32),
            pltpu.VMEM((CHUNK,), jnp.int32),
            pltpu.VMEM((CHUNK,), jnp.int32),
            pltpu.VMEM((HALF, 128), jnp.int32),
            pltpu.VMEM((HALF, 128), jnp.int32),
            pltpu.VMEM((BIGRAM_DIM, CHUNK), jnp.float32),
            pltpu.SemaphoreType.DMA,
            pltpu.SemaphoreType.DMA,
        ],
    )
    def _sc_gather(x_hbm, prev_hbm, t_hbm, out_hbm,
                   x_v, prev_v, h_v, idx_v, rawa_v, rawb_v, ex_v, sema, semb):
        base = (lax.axis_index("s") * NC + lax.axis_index("c")) * CHUNK
        pltpu.sync_copy(x_hbm.at[pl.ds(base, CHUNK)], x_v)
        pltpu.sync_copy(prev_hbm.at[pl.ds(base, CHUNK)], prev_v)

        def hashes(i, _):
            xa = x_v[pl.ds(i * 16, 16)]
            pa = prev_v[pl.ds(i * 16, 16)]
            h = (pa * hash_mul_prev + xa * hash_mul) % BIGRAM_VOCAB
            h_v[pl.ds(i * 16, 16)] = h
            idx_v[pl.ds(i * 16, 16)] = ((h >> 16) << 13) | (h & 8191)
            return ()

        lax.fori_loop(0, CHUNK // 16, hashes, ())

        bufs = [rawa_v, rawb_v]
        sems = [sema, semb]
        lane = lax.iota(jnp.int32, 16)

        def fire(s):
            return [pltpu.async_copy(
                t_hbm.at[idx_v.at[pl.ds(s * HALF + c * 128, 128)]],
                bufs[s].at[pl.ds(c * 128, 128)], sems[s])
                for c in range(HALF // 128)]

        def extract(s):
            def body(g, _):
                hv = h_v[pl.ds(s * HALF + g * 16, 16)]
                sub = ((hv >> 13) & 3) * 32
                is_hi = ((hv >> 13) & 7) < 4
                row = lane + g * 16
                for w in range(BIGRAM_DIM):
                    w32 = plsc.load_gather(bufs[s], [row, sub + w])
                    bits = jnp.where(is_hi, w32 & jnp.int32(-65536), w32 << 16)
                    ex_v[w, pl.ds(s * HALF + g * 16, 16)] = plsc.bitcast(
                        bits, jnp.float32)
                return ()

            lax.fori_loop(0, HALF // 16, body, ())

        inflight = fire(0)
        nxt = fire(1)
        for cp in inflight:
            cp.wait()
        extract(0)
        for cp in nxt:
            cp.wait()
        extract(1)
        pltpu.sync_copy(ex_v, out_hbm.at[:, pl.ds(base, CHUNK)])

    return _sc_gather


_sc_gather_1 = _make_sc_gather(hash_mul=1, hash_mul_prev=1024)
_sc_gather_2 = _make_sc_gather(hash_mul=31, hash_mul_prev=1)


def _tc_transpose_body(t_ref, o_ref):
    z = jnp.concatenate(
        [t_ref[:, C_BLK * s:C_BLK * s + C_BLK] for s in range(8)], axis=0)
    bits = lax.bitcast_convert_type(
        z.astype(jnp.bfloat16), jnp.uint16).astype(jnp.uint32)
    zi = (bits[:128] << 16) | bits[128:]
    o_ref[...] = lax.bitcast_convert_type(zi, jnp.int32).T


def _tc_transpose(t_t):
    return pl.pallas_call(
        _tc_transpose_body,
        grid=(N_TBLK,),
        in_specs=[pl.BlockSpec((BIGRAM_DIM, W_BLK), lambda i: (0, i))],
        out_specs=pl.BlockSpec((C_BLK, 128), lambda i: (i, 0)),
        out_shape=jax.ShapeDtypeStruct((ROWS, 128), jnp.int32),
    )(t_t)


M_BLK = 2048


def _tc_matmul_body(gate_ref, e1_ref, e2_ref, w_ref, o_ref):
    g = jax.nn.sigmoid(gate_ref[0])
    w1 = w_ref[:, :BIGRAM_DIM] * g
    w2 = w_ref[:, BIGRAM_DIM:] * (1.0 - g)
    acc = lax.dot_general(e1_ref[...], w1, (((0,), (1,)), ((), ())),
                          preferred_element_type=jnp.float32)
    acc += lax.dot_general(e2_ref[...], w2, (((0,), (1,)), ((), ())),
                           preferred_element_type=jnp.float32)
    o_ref[...] = acc


def _tc_matmul(gate_flat, e1, e2, proj_w):
    return pl.pallas_call(
        _tc_matmul_body,
        grid=(B_TOTAL // M_BLK,),
        in_specs=[
            pl.BlockSpec(memory_space=pltpu.SMEM),
            pl.BlockSpec((BIGRAM_DIM, M_BLK), lambda i: (0, i)),
            pl.BlockSpec((BIGRAM_DIM, M_BLK), lambda i: (0, i)),
            pl.BlockSpec((MODEL_DIM, 2 * BIGRAM_DIM), lambda i: (0, 0)),
        ],
        out_specs=pl.BlockSpec((M_BLK, MODEL_DIM), lambda i: (i, 0)),
        out_shape=jax.ShapeDtypeStruct((B_TOTAL, MODEL_DIM), jnp.float32),
    )(gate_flat, e1, e2, proj_w)


def kernel(x, embed1, embed2, proj_w, gate):
    batch, seqlen = x.shape
    x_flat = x.reshape(-1)
    prev_flat = jnp.pad(x[:, :-1], ((0, 0), (1, 0)), constant_values=0).reshape(-1)
    t1 = _tc_transpose(embed1.T)
    e1 = _sc_gather_1(x_flat, prev_flat, t1)
    t2 = _tc_transpose(embed2.T)
    e2 = _sc_gather_2(x_flat, prev_flat, t2)
    y = _tc_matmul(gate.reshape(-1), e1, e2, proj_w)
    return y.reshape(batch, seqlen, MODEL_DIM)


# speedup vs baseline: 5.0091x; 1.0019x over previous
import functools

import jax
import jax.numpy as jnp
from jax import lax
from jax.experimental import pallas as pl
from jax.experimental.pallas import tpu as pltpu
from jax.experimental.pallas import tpu_sc as plsc

VOCAB = 100000
BIGRAM_VOCAB = 1000000
BIGRAM_DIM = 32
MODEL_DIM = 768

NC = 2
NS = 16
NW = NC * NS
B_TOTAL = 4 * 4096
CHUNK = B_TOTAL // NW
HALF = CHUNK // 2
W_BLK = 131072
C_BLK = W_BLK // 8
N_TBLK = -(-BIGRAM_VOCAB // W_BLK)
ROWS = N_TBLK * C_BLK

_sc_mesh = plsc.VectorSubcoreMesh(
    core_axis_name="c", subcore_axis_name="s", num_cores=NC, num_subcores=NS
)


def _make_sc_gather(hash_mul, hash_mul_prev):
    @functools.partial(
        pl.kernel,
        out_type=jax.ShapeDtypeStruct((BIGRAM_DIM, B_TOTAL), jnp.float32),
        mesh=_sc_mesh,
        compiler_params=pltpu.CompilerParams(
            use_tc_tiling_on_sc=True, needs_layout_passes=False),
        scratch_types=[
            pltpu.VMEM((CHUNK,), jnp.int32),
            pltpu.VMEM((CHUNK,), jnp.int32),
            pltpu.VMEM((CHUNK,), jnp.int32),
            pltpu.VMEM((CHUNK,), jnp.int32),
            pltpu.VMEM((HALF, 128), jnp.int32),
            pltpu.VMEM((HALF, 128), jnp.int32),
            pltpu.VMEM((BIGRAM_DIM, CHUNK), jnp.float32),
            pltpu.SemaphoreType.DMA,
            pltpu.SemaphoreType.DMA,
        ],
    )
    def _sc_gather(x_hbm, prev_hbm, t_hbm, out_hbm,
                   x_v, prev_v, h_v, idx_v, rawa_v, rawb_v, ex_v, sema, semb):
        base = (lax.axis_index("s") * NC + lax.axis_index("c")) * CHUNK
        pltpu.sync_copy(x_hbm.at[pl.ds(base, CHUNK)], x_v)
        pltpu.sync_copy(prev_hbm.at[pl.ds(base, CHUNK)], prev_v)

        def hashes(i, _):
            xa = x_v[pl.ds(i * 16, 16)]
            pa = prev_v[pl.ds(i * 16, 16)]
            h = (pa * hash_mul_prev + xa * hash_mul) % BIGRAM_VOCAB
            h_v[pl.ds(i * 16, 16)] = h
            idx_v[pl.ds(i * 16, 16)] = ((h >> 17) << 14) | (h & 16383)
            return ()

        lax.fori_loop(0, CHUNK // 16, hashes, ())

        bufs = [rawa_v, rawb_v]
        sems = [sema, semb]
        lane = lax.iota(jnp.int32, 16)

        def fire(s):
            return [pltpu.async_copy(
                t_hbm.at[idx_v.at[pl.ds(s * HALF + c * 128, 128)]],
                bufs[s].at[pl.ds(c * 128, 128)], sems[s])
                for c in range(HALF // 128)]

        def extract(s):
            def body(g, _):
                hv = h_v[pl.ds(s * HALF + g * 16, 16)]
                sub = ((hv >> 14) & 3) * 32
                is_hi = ((hv >> 14) & 7) < 4
                row = lane + g * 16
                for w in range(BIGRAM_DIM):
                    w32 = plsc.load_gather(bufs[s], [row, sub + w])
                    bits = jnp.where(is_hi, w32 & jnp.int32(-65536), w32 << 16)
                    ex_v[w, pl.ds(s * HALF + g * 16, 16)] = plsc.bitcast(
                        bits, jnp.float32)
                return ()

            lax.fori_loop(0, HALF // 16, body, ())

        inflight = fire(0)
        nxt = fire(1)
        for cp in inflight:
            cp.wait()
        extract(0)
        for cp in nxt:
            cp.wait()
        extract(1)
        pltpu.sync_copy(ex_v, out_hbm.at[:, pl.ds(base, CHUNK)])

    return _sc_gather


_sc_gather_1 = _make_sc_gather(hash_mul=1, hash_mul_prev=1024)
_sc_gather_2 = _make_sc_gather(hash_mul=31, hash_mul_prev=1)


def _tc_transpose_body(t_ref, o_ref):
    z = jnp.concatenate(
        [t_ref[:, C_BLK * s:C_BLK * s + C_BLK] for s in range(8)], axis=0)
    bits = lax.bitcast_convert_type(
        z.astype(jnp.bfloat16), jnp.uint16).astype(jnp.uint32)
    zi = (bits[:128] << 16) | bits[128:]
    o_ref[...] = lax.bitcast_convert_type(zi, jnp.int32).T


def _tc_transpose(t_t):
    return pl.pallas_call(
        _tc_transpose_body,
        grid=(N_TBLK,),
        in_specs=[pl.BlockSpec((BIGRAM_DIM, W_BLK), lambda i: (0, i))],
        out_specs=pl.BlockSpec((C_BLK, 128), lambda i: (i, 0)),
        out_shape=jax.ShapeDtypeStruct((ROWS, 128), jnp.int32),
    )(t_t)


M_BLK = 2048


def _tc_matmul_body(gate_ref, e1_ref, e2_ref, w_ref, o_ref):
    g = jax.nn.sigmoid(gate_ref[0])
    w1 = w_ref[:, :BIGRAM_DIM] * g
    w2 = w_ref[:, BIGRAM_DIM:] * (1.0 - g)
    acc = lax.dot_general(e1_ref[...], w1, (((0,), (1,)), ((), ())),
                          preferred_element_type=jnp.float32)
    acc += lax.dot_general(e2_ref[...], w2, (((0,), (1,)), ((), ())),
                           preferred_element_type=jnp.float32)
    o_ref[...] = acc


def _tc_matmul(gate_flat, e1, e2, proj_w):
    return pl.pallas_call(
        _tc_matmul_body,
        grid=(B_TOTAL // M_BLK,),
        in_specs=[
            pl.BlockSpec(memory_space=pltpu.SMEM),
            pl.BlockSpec((BIGRAM_DIM, M_BLK), lambda i: (0, i)),
            pl.BlockSpec((BIGRAM_DIM, M_BLK), lambda i: (0, i)),
            pl.BlockSpec((MODEL_DIM, 2 * BIGRAM_DIM), lambda i: (0, 0)),
        ],
        out_specs=pl.BlockSpec((M_BLK, MODEL_DIM), lambda i: (i, 0)),
        out_shape=jax.ShapeDtypeStruct((B_TOTAL, MODEL_DIM), jnp.float32),
    )(gate_flat, e1, e2, proj_w)


def kernel(x, embed1, embed2, proj_w, gate):
    batch, seqlen = x.shape
    x_flat = x.reshape(-1)
    prev_flat = jnp.pad(x[:, :-1], ((0, 0), (1, 0)), constant_values=0).reshape(-1)
    t1 = _tc_transpose(embed1.T)
    e1 = _sc_gather_1(x_flat, prev_flat, t1)
    t2 = _tc_transpose(embed2.T)
    e2 = _sc_gather_2(x_flat, prev_flat, t2)
    y = _tc_matmul(gate.reshape(-1), e1, e2, proj_w)
    return y.reshape(batch, seqlen, MODEL_DIM)
